# baseline (speedup 1.0000x reference)
import functools

import jax
import jax.numpy as jnp
from jax import lax
from jax.experimental import pallas as pl
from jax.experimental.pallas import tpu as pltpu
from jax.experimental.pallas import tpu_sc as plsc

EMB = 64
MAX_LEN = 200
BATCH = 4096
ROWPAIR = 128

NC = 2
NS = 16
NW = NC * NS
BW = BATCH // NW
NBUF = 3
NOBUF = 2
LANES = 16
ER = EMB // 8
GRP = 6

_mesh = plsc.VectorSubcoreMesh(core_axis_name="c", subcore_axis_name="s")


@functools.partial(
    pl.kernel,
    mesh=_mesh,
    out_type=jax.ShapeDtypeStruct((MAX_LEN, ER, NW, 8, 128), jnp.float32),
    compiler_params=pltpu.CompilerParams(use_tc_tiling_on_sc=False,
                                         needs_layout_passes=False),
    scratch_types=[
        pltpu.VMEM((MAX_LEN, BW), jnp.int32),
        pltpu.VMEM((MAX_LEN, EMB), jnp.float32),
        [pltpu.VMEM((BW, ROWPAIR), jnp.float32)] * NBUF,
        [pltpu.VMEM((EMB, BW), jnp.float32)] * NOBUF,
        [pltpu.VMEM((BW,), jnp.int32)] * NBUF,
        [pltpu.SemaphoreType.DMA] * NBUF,
        [pltpu.SemaphoreType.DMA] * NOBUF,
    ],
)
def _emb_lookup(tok_hbm, table_hbm, pos_hbm, out_hbm,
                idx_v, pos_v, gbufs, obufs, ibufs, gsems, osems):
    wid = lax.axis_index("s") * NC + lax.axis_index("c")

    pltpu.sync_copy(pos_hbm, pos_v)
    pltpu.sync_copy(tok_hbm.at[:, pl.ds(wid * BW, BW)], idx_v)

    iota = lax.broadcasted_iota(jnp.int32, (LANES,), 0)
    rows_c = [iota + (b0 * LANES) for b0 in range(BW // LANES)]

    def prep_gather(l, slot):
        for b0 in range(BW // LANES):
            tv = idx_v[l, pl.ds(b0 * LANES, LANES)]
            ibufs[slot][pl.ds(b0 * LANES, LANES)] = tv
        pltpu.make_async_copy(table_hbm.at[ibufs[slot]], gbufs[slot],
                              gsems[slot]).start()

    def gather_wait(slot):
        pltpu.make_async_copy(table_hbm.at[ibufs[slot]], gbufs[slot],
                              gsems[slot]).wait()

    def out_dma(l, oslot, er):
        dst = out_hbm.at[l, er, wid]
        return pltpu.make_async_copy(obufs[oslot].at[pl.ds(er * 8, 8)],
                                     dst, osems[oslot])

    def substep(l, slot, oslot):
        gather_wait(slot)
        gbuf = gbufs[slot]
        obuf = obufs[oslot]

        @pl.when(l >= NOBUF)
        def _():
            for er in range(ER):
                out_dma(l - NOBUF, oslot, er).wait()

        lvec = jnp.broadcast_to(l, (LANES,))

        def trans_k(k, carry):
            diag = (iota + k) & 15
            for e0 in range(0, EMB, LANES):
                cols = diag + e0
                p = plsc.load_gather(pos_v, [lvec, cols])
                for b0 in range(BW // LANES):
                    v = plsc.load_gather(gbuf, [rows_c[b0], cols])
                    plsc.store_scatter(obuf, [cols, rows_c[b0]], v + p)
            return carry

        lax.fori_loop(0, LANES, trans_k, 0)

        @pl.when(l + NBUF < MAX_LEN)
        def _():
            prep_gather(l + NBUF, slot)

        for er in range(ER):
            out_dma(l, oslot, er).start()

    for s in range(NBUF):
        prep_gather(s, s)

    def body(k, carry):
        for s in range(GRP):
            substep(k * GRP + s, s % NBUF, s % NOBUF)
        return carry

    lax.fori_loop(0, MAX_LEN // GRP, body, 0)
    for l in range(MAX_LEN - MAX_LEN % GRP, MAX_LEN):
        substep(l, l % NBUF, l % NOBUF)

    for l in range(MAX_LEN - NOBUF, MAX_LEN):
        for er in range(ER):
            out_dma(l, l % NOBUF, er).wait()


def kernel(tokens, token_table, pos_emb):
    tok_t = tokens.T
    tab_p = jnp.pad(token_table, ((0, 0), (0, ROWPAIR - EMB)))
    o5 = _emb_lookup(tok_t, tab_p, pos_emb)
    return o5.transpose(2, 4, 0, 1, 3).reshape(BATCH, MAX_LEN, EMB)


# speedup vs baseline: 1.0404x; 1.0404x over previous
import functools

import jax
import jax.numpy as jnp
from jax import lax
from jax.experimental import pallas as pl
from jax.experimental.pallas import tpu as pltpu
from jax.experimental.pallas import tpu_sc as plsc

EMB = 64
MAX_LEN = 200
BATCH = 4096
ROWPAIR = 128

NC = 2
NS = 16
NW = NC * NS
BW = BATCH // NW
NBUF = 4
NOBUF = 2
LANES = 16
ER = EMB // 8
GRP = 4

_mesh = plsc.VectorSubcoreMesh(core_axis_name="c", subcore_axis_name="s")


@functools.partial(
    pl.kernel,
    mesh=_mesh,
    out_type=jax.ShapeDtypeStruct((MAX_LEN, ER, NW, 8, 128), jnp.float32),
    compiler_params=pltpu.CompilerParams(use_tc_tiling_on_sc=False,
                                         needs_layout_passes=False),
    scratch_types=[
        pltpu.VMEM((MAX_LEN, BW), jnp.int32),
        pltpu.VMEM((MAX_LEN, EMB), jnp.float32),
        [pltpu.VMEM((BW, ROWPAIR), jnp.float32)] * NBUF,
        [pltpu.VMEM((ER, 8, BW), jnp.float32)] * NOBUF,
        [pltpu.SemaphoreType.DMA] * NBUF,
        [pltpu.SemaphoreType.DMA] * NOBUF,
    ],
)
def _emb_lookup(tok_hbm, table_hbm, pos_hbm, out_hbm,
                idx_v, pos_v, gbufs, obufs, gsems, osems):
    wid = lax.axis_index("s") * NC + lax.axis_index("c")

    pltpu.sync_copy(pos_hbm, pos_v)
    pltpu.sync_copy(tok_hbm.at[:, pl.ds(wid * BW, BW)], idx_v)

    iota = lax.broadcasted_iota(jnp.int32, (LANES,), 0)
    rows_c = [iota + (b0 * LANES) for b0 in range(BW // LANES)]

    def gather(l, slot):
        return pltpu.make_async_copy(table_hbm.at[idx_v.at[l]], gbufs[slot],
                                     gsems[slot])

    def out_dma(l, oslot):
        return pltpu.make_async_copy(obufs[oslot], out_hbm.at[l, :, wid],
                                     osems[oslot])

    def substep(l, slot, oslot):
        gather(l, slot).wait()
        gbuf = gbufs[slot]
        obuf = obufs[oslot]

        @pl.when(l >= NOBUF)
        def _():
            out_dma(l - NOBUF, oslot).wait()

        lvec = jnp.broadcast_to(l, (LANES,))

        def trans_k(k, carry):
            diag = (iota + k) & 15
            dhi = lax.shift_right_logical(diag, 3)
            dlo = diag & 7
            for e0 in range(0, EMB, LANES):
                cols = diag + e0
                p = plsc.load_gather(pos_v, [lvec, cols])
                ehi = dhi + (e0 // 8)
                for b0 in range(BW // LANES):
                    v = plsc.load_gather(gbuf, [rows_c[b0], cols])
                    plsc.store_scatter(obuf, [ehi, dlo, rows_c[b0]], v + p)
            return carry

        lax.fori_loop(0, LANES, trans_k, 0)

        @pl.when(l + NBUF < MAX_LEN)
        def _():
            gather(l + NBUF, slot).start()

        out_dma(l, oslot).start()

    for s in range(NBUF):
        gather(s, s).start()

    def body(k, carry):
        for s in range(GRP):
            substep(k * GRP + s, s % NBUF, s % NOBUF)
        return carry

    lax.fori_loop(0, MAX_LEN // GRP, body, 0)
    for l in range(MAX_LEN - MAX_LEN % GRP, MAX_LEN):
        substep(l, l % NBUF, l % NOBUF)

    for l in range(MAX_LEN - NOBUF, MAX_LEN):
        out_dma(l, l % NOBUF).wait()


def kernel(tokens, token_table, pos_emb):
    tok_t = tokens.T
    tab_p = jnp.pad(token_table, ((0, 0), (0, ROWPAIR - EMB)))
    o5 = _emb_lookup(tok_t, tab_p, pos_emb)
    return o5.transpose(2, 4, 0, 1, 3).reshape(BATCH, MAX_LEN, EMB)


# speedup vs baseline: 1.4439x; 1.3879x over previous
import functools

import jax
import jax.numpy as jnp
from jax import lax
from jax.experimental import pallas as pl
from jax.experimental.pallas import tpu as pltpu
from jax.experimental.pallas import tpu_sc as plsc

EMB = 64
MAX_LEN = 200
BATCH = 4096
ROWPAIR = 128

NC = 2
NS = 16
NW = NC * NS
BW = BATCH // NW
NBUF = 4
NOBUF = 2
LANES = 16
ER = EMB // 8
GRP = 4

_mesh = plsc.VectorSubcoreMesh(core_axis_name="c", subcore_axis_name="s")


@functools.partial(
    pl.kernel,
    mesh=_mesh,
    out_type=jax.ShapeDtypeStruct((MAX_LEN, ER, NW, 8, 128), jnp.float32),
    compiler_params=pltpu.CompilerParams(use_tc_tiling_on_sc=False,
                                         needs_layout_passes=False),
    scratch_types=[
        pltpu.VMEM((MAX_LEN, BW), jnp.int32),
        pltpu.VMEM((MAX_LEN, EMB), jnp.float32),
        [pltpu.VMEM((BW, ROWPAIR), jnp.float32)] * NBUF,
        [pltpu.VMEM((ER, 8, BW), jnp.float32)] * NOBUF,
        [pltpu.SemaphoreType.DMA] * NBUF,
        [pltpu.SemaphoreType.DMA] * NOBUF,
    ],
)
def _emb_lookup(tok_hbm, table_hbm, pos_hbm, out_hbm,
                idx_v, pos_v, gbufs, obufs, gsems, osems):
    wid = lax.axis_index("s") * NC + lax.axis_index("c")

    pltpu.sync_copy(pos_hbm, pos_v)
    pltpu.sync_copy(tok_hbm.at[:, pl.ds(wid * BW, BW)], idx_v)

    iota = lax.broadcasted_iota(jnp.int32, (LANES,), 0)
    zero16 = jnp.broadcast_to(0, (LANES,))
    rows_c = [iota + (b0 * LANES) for b0 in range(BW // LANES)]
    rows_g = [lax.shift_left(iota + (b0 * LANES), 7) for b0 in range(BW // LANES)]

    def gather(l, slot):
        return pltpu.make_async_copy(table_hbm.at[idx_v.at[l]], gbufs[slot],
                                     gsems[slot])

    def out_dma(l, oslot):
        return pltpu.make_async_copy(obufs[oslot], out_hbm.at[l, :, wid],
                                     osems[oslot])

    def substep(l, slot, oslot):
        gather(l, slot).wait()
        gbuf = gbufs[slot]
        obuf = obufs[oslot]

        @pl.when(l >= NOBUF)
        def _():
            out_dma(l - NOBUF, oslot).wait()

        lbase = jnp.broadcast_to(l * EMB, (LANES,))

        def trans_k(k, carry):
            diag = (iota + k) & 15
            for e0 in range(0, EMB, LANES):
                cols = diag + e0
                p = plsc.load_gather(pos_v, [zero16, lbase + cols])
                cshift = lax.shift_left(cols, 7)
                vs = [plsc.load_gather(gbuf, [zero16, rows_g[b0] + cols])
                      for b0 in range(BW // LANES)]
                for b0 in range(BW // LANES):
                    plsc.store_scatter(obuf, [zero16, zero16, cshift + rows_c[b0]],
                                       vs[b0] + p)
            return carry

        lax.fori_loop(0, LANES, trans_k, 0)

        @pl.when(l + NBUF < MAX_LEN)
        def _():
            gather(l + NBUF, slot).start()

        out_dma(l, oslot).start()

    for s in range(NBUF):
        gather(s, s).start()

    def body(k, carry):
        for s in range(GRP):
            substep(k * GRP + s, s % NBUF, s % NOBUF)
        return carry

    lax.fori_loop(0, MAX_LEN // GRP, body, 0)
    for l in range(MAX_LEN - MAX_LEN % GRP, MAX_LEN):
        substep(l, l % NBUF, l % NOBUF)

    for l in range(MAX_LEN - NOBUF, MAX_LEN):
        out_dma(l, l % NOBUF).wait()


def kernel(tokens, token_table, pos_emb):
    tok_t = tokens.T
    tab_p = jnp.pad(token_table, ((0, 0), (0, ROWPAIR - EMB)))
    o5 = _emb_lookup(tok_t, tab_p, pos_emb)
    return o5.transpose(2, 4, 0, 1, 3).reshape(BATCH, MAX_LEN, EMB)
